# Initial kernel scaffold; baseline (speedup 1.0000x reference)
import jax
import jax.numpy as jnp
from jax.experimental import pallas as pl


def kernel(x, edge_index, c1_init, c1_root, c1_bias, c2_init, c2_root, c2_bias):
    raise NotImplementedError("write your pallas kernel here")


# speedup vs baseline: 63.9299x; 63.9299x over previous
import functools

import jax
import jax.numpy as jnp
from jax import lax
from jax.experimental import pallas as pl
from jax.experimental.pallas import tpu as pltpu
from jax.experimental.pallas import tpu_sc as plsc

_NC = 2
_NS = 16
_NW = _NC * _NS
_CH = 80
_RT = 400


def _propagate(n, e, d, gather):
  epw = e // _NW
  nchunks = epw // _CH
  rpt8 = (n // _NS) // 8 * 8
  tail = n - _NS * rpt8
  mesh = plsc.VectorSubcoreMesh(core_axis_name="c", subcore_axis_name="s",
                                num_cores=_NC, num_subcores=_NS)

  @functools.partial(
      pl.kernel,
      out_type=jax.ShapeDtypeStruct((_NC, n, d), jnp.float32),
      mesh=mesh,
      compiler_params=pltpu.CompilerParams(use_tc_tiling_on_sc=False),
      scratch_types=[
          pltpu.VMEM((_CH,), jnp.int32),
          pltpu.VMEM((_CH,), jnp.int32),
          pltpu.VMEM((_CH, d), jnp.float32),
          pltpu.VMEM_SHARED((n, d), jnp.float32),
          pltpu.SemaphoreType.DMA,
      ],
  )
  def prop(row1d, col1d, table, zeros, ones, out, ridx, cidx, rows_v, acc, sem):
    c = lax.axis_index("c")
    s = lax.axis_index("s")
    wid = s * _NC + c
    pltpu.sync_copy(zeros.at[pl.ds(s * rpt8, rpt8)],
                    acc.at[pl.ds(s * rpt8, rpt8)])

    @pl.when(s == _NS - 1)
    def _():
      pltpu.sync_copy(zeros.at[pl.ds(_NS * rpt8, tail)],
                      acc.at[pl.ds(_NS * rpt8, tail)])

    if not gather:
      pltpu.sync_copy(ones, rows_v)
    plsc.subcore_barrier()
    base = wid * nchunks

    def body(j, carry):
      t = (base + j) * _CH
      pltpu.sync_copy(col1d.at[pl.ds(t, _CH)], cidx)
      if gather:
        pltpu.sync_copy(row1d.at[pl.ds(t, _CH)], ridx)
        pltpu.async_copy(table.at[ridx], rows_v, sem).wait()
      pltpu.sync_copy(rows_v, acc.at[cidx], add=True)
      return carry

    lax.fori_loop(0, nchunks, body, 0)
    plsc.subcore_barrier()
    pltpu.sync_copy(acc.at[pl.ds(s * rpt8, rpt8)],
                    out.at[c, pl.ds(s * rpt8, rpt8)])

    @pl.when(s == _NS - 1)
    def _():
      pltpu.sync_copy(acc.at[pl.ds(_NS * rpt8, tail)],
                      out.at[c, pl.ds(_NS * rpt8, tail)])

  return prop


def _dinv_of(dp):
  deg = dp[0][:, 0:1] + dp[1][:, 0:1]
  return jnp.where(deg > 0, lax.rsqrt(jnp.maximum(deg, 1e-12)), 0.0)


def _tc_input(x, w1, v1):
  n, f = x.shape
  kd = w1.shape[1]

  def body(x_ref, w_ref, v_ref, z_ref, r_ref):
    xb = x_ref[...]
    rn = xb / jnp.maximum(jnp.sum(xb, axis=1, keepdims=True), 1e-8)
    z_ref[...] = jnp.dot(rn, w_ref[...], preferred_element_type=jnp.float32)
    r_ref[...] = jnp.dot(rn, v_ref[...], preferred_element_type=jnp.float32)

  return pl.pallas_call(
      body,
      grid=(n // _RT,),
      in_specs=[pl.BlockSpec((_RT, f), lambda i: (i, 0)),
                pl.BlockSpec((f, kd), lambda i: (0, 0)),
                pl.BlockSpec((f, kd), lambda i: (0, 0))],
      out_specs=[pl.BlockSpec((_RT, kd), lambda i: (i, 0)),
                 pl.BlockSpec((_RT, kd), lambda i: (i, 0))],
      out_shape=[jax.ShapeDtypeStruct((n, kd), jnp.float32),
                 jax.ShapeDtypeStruct((n, kd), jnp.float32)],
  )(x, w1, v1)


def _tc_scale(degp, z):
  n, kd = z.shape

  def body(dp_ref, z_ref, o_ref):
    o_ref[...] = z_ref[...] * _dinv_of(dp_ref[...])

  return pl.pallas_call(
      body,
      grid=(n // _RT,),
      in_specs=[pl.BlockSpec((_NC, _RT, 16), lambda i: (0, i, 0)),
                pl.BlockSpec((_RT, kd), lambda i: (i, 0))],
      out_specs=pl.BlockSpec((_RT, kd), lambda i: (i, 0)),
      out_shape=jax.ShapeDtypeStruct((n, kd), jnp.float32),
  )(degp, z)


def _tc_combine1(degp, pp, r1, b1, v2):
  n = r1.shape[0]
  kd = r1.shape[1]
  hid = kd // 2
  dout = v2.shape[1]

  def body(dp_ref, pp_ref, r1_ref, b1_ref, v2_ref, hs_ref, hv2_ref):
    dinv = _dinv_of(dp_ref[...])
    agg = (pp_ref[0] + pp_ref[1]) * dinv
    u = agg + r1_ref[...] + b1_ref[...]
    h = 0.5 * (jax.nn.relu(u[:, :hid]) + jax.nn.relu(u[:, hid:]))
    hs_ref[...] = h * dinv
    hv2_ref[...] = jnp.dot(h, v2_ref[...], preferred_element_type=jnp.float32)

  return pl.pallas_call(
      body,
      grid=(n // _RT,),
      in_specs=[pl.BlockSpec((_NC, _RT, 16), lambda i: (0, i, 0)),
                pl.BlockSpec((_NC, _RT, kd), lambda i: (0, i, 0)),
                pl.BlockSpec((_RT, kd), lambda i: (i, 0)),
                pl.BlockSpec((1, kd), lambda i: (0, 0)),
                pl.BlockSpec((hid, dout), lambda i: (0, 0))],
      out_specs=[pl.BlockSpec((_RT, hid), lambda i: (i, 0)),
                 pl.BlockSpec((_RT, dout), lambda i: (i, 0))],
      out_shape=[jax.ShapeDtypeStruct((n, hid), jnp.float32),
                 jax.ShapeDtypeStruct((n, dout), jnp.float32)],
  )(degp, pp, r1, b1, v2)


def _tc_combine2(degp, qp, hv2, w2, b2):
  n, dout = hv2.shape
  hid = w2.shape[0]

  def body(dp_ref, qp_ref, hv2_ref, w2_ref, b2_ref, o_ref):
    dinv = _dinv_of(dp_ref[...])
    aggh = (qp_ref[0] + qp_ref[1]) * dinv
    o_ref[...] = jax.nn.relu(
        jnp.dot(aggh, w2_ref[...], preferred_element_type=jnp.float32)
        + hv2_ref[...] + b2_ref[...])

  return pl.pallas_call(
      body,
      grid=(n // _RT,),
      in_specs=[pl.BlockSpec((_NC, _RT, 16), lambda i: (0, i, 0)),
                pl.BlockSpec((_NC, _RT, hid), lambda i: (0, i, 0)),
                pl.BlockSpec((_RT, dout), lambda i: (i, 0)),
                pl.BlockSpec((hid, dout), lambda i: (0, 0)),
                pl.BlockSpec((1, dout), lambda i: (0, 0))],
      out_specs=pl.BlockSpec((_RT, dout), lambda i: (i, 0)),
      out_shape=jax.ShapeDtypeStruct((n, dout), jnp.float32),
  )(degp, qp, hv2, w2, b2)


def kernel(x, edge_index, c1_init, c1_root, c1_bias, c2_init, c2_root, c2_bias):
  n, f = x.shape
  e = edge_index.shape[1]
  k1, hid = c1_init.shape[0], c1_init.shape[2]
  dout = c2_init.shape[2]
  kd = k1 * hid

  row1d = edge_index[0]
  col1d = edge_index[1]
  w1 = jnp.transpose(c1_init, (1, 0, 2)).reshape(f, kd)
  v1 = jnp.transpose(c1_root[0], (1, 0, 2)).reshape(f, kd)
  b1 = jnp.transpose(c1_bias[0], (1, 0, 2)).reshape(1, kd)
  w2 = c2_init[0]
  v2 = c2_root[0, 0]
  b2 = c2_bias[0, 0]
  zeros16 = jnp.zeros((n, 16), jnp.float32)
  zeros32 = jnp.zeros((n, kd), jnp.float32)
  ones16 = jnp.ones((_CH, 16), jnp.float32)

  z, r1 = _tc_input(x, w1, v1)
  degp = _propagate(n, e, 16, False)(row1d, col1d, zeros16, zeros16, ones16)
  zs = _tc_scale(degp, z)
  pp = _propagate(n, e, kd, True)(row1d, col1d, zs, zeros32, ones16)
  hs, hv2 = _tc_combine1(degp, pp, r1, b1, v2)
  qp = _propagate(n, e, hid, True)(row1d, col1d, hs, zeros16, ones16)
  return _tc_combine2(degp, qp, hv2, w2, b2)


# speedup vs baseline: 174.6604x; 2.7321x over previous
import functools

import jax
import jax.numpy as jnp
from jax import lax
from jax.experimental import pallas as pl
from jax.experimental.pallas import tpu as pltpu
from jax.experimental.pallas import tpu_sc as plsc

_NC = 2
_NS = 16
_NW = _NC * _NS
_CH = 80
_RT = 400


def _propagate(n, e, d, gather):
  epw = e // _NW
  nchunks = epw // _CH
  K = 5
  nblocks = nchunks // K
  rpt8 = (n // _NS) // 8 * 8
  tail = n - _NS * rpt8
  mesh = plsc.VectorSubcoreMesh(core_axis_name="c", subcore_axis_name="s",
                                num_cores=_NC, num_subcores=_NS)

  @functools.partial(
      pl.kernel,
      out_type=jax.ShapeDtypeStruct((_NC, n, d), jnp.float32),
      mesh=mesh,
      compiler_params=pltpu.CompilerParams(use_tc_tiling_on_sc=False),
      scratch_types=[
          pltpu.VMEM((nchunks, _CH), jnp.int32),
          pltpu.VMEM((nchunks, _CH), jnp.int32),
          pltpu.VMEM((2, K, _CH, d), jnp.float32),
          pltpu.VMEM_SHARED((n, d), jnp.float32),
          pltpu.SemaphoreType.DMA,
          pltpu.SemaphoreType.DMA,
      ],
  )
  def prop(row2d, col2d, table, zeros, ones, out,
           ridx, cidx, bufs, acc, gsem, ssem):
    c = lax.axis_index("c")
    s = lax.axis_index("s")
    wid = s * _NC + c
    pltpu.sync_copy(col2d.at[pl.ds(wid * nchunks, nchunks)], cidx)
    if gather:
      pltpu.sync_copy(row2d.at[pl.ds(wid * nchunks, nchunks)], ridx)
    pltpu.sync_copy(zeros.at[pl.ds(s * rpt8, rpt8)],
                    acc.at[pl.ds(s * rpt8, rpt8)])

    @pl.when(s == _NS - 1)
    def _():
      pltpu.sync_copy(zeros.at[pl.ds(_NS * rpt8, tail)],
                      acc.at[pl.ds(_NS * rpt8, tail)])

    if not gather:
      for b in range(K):
        pltpu.sync_copy(ones, bufs.at[0, b])
    plsc.subcore_barrier()

    def issue_gather(j, g, b):
      pltpu.async_copy(table.at[ridx.at[j]], bufs.at[g, b], gsem)

    def wait_gather(g, b):
      pltpu.make_async_copy(table.at[pl.ds(0, _CH)], bufs.at[g, b], gsem).wait()

    def issue_scatter(j, g, b):
      pltpu.async_copy(bufs.at[g, b], acc.at[cidx.at[j]], ssem, add=True)

    def wait_scatter(g, b):
      pltpu.make_async_copy(table.at[pl.ds(0, _CH)], bufs.at[g, b], ssem).wait()

    if gather:
      def do_block(bi, g, drain_prev, prefetch):
        if drain_prev:
          for b in range(K):
            wait_scatter(1 - g, b)
        for b in range(K):
          wait_gather(g, b)
        if prefetch:
          for b in range(K):
            issue_gather((bi + 1) * K + b, 1 - g, b)
        for b in range(K):
          issue_scatter(bi * K + b, g, b)

      for b in range(K):
        issue_gather(b, 0, b)
      do_block(0, 0, drain_prev=False, prefetch=True)

      def body(i, carry):
        do_block(2 * i + 1, 1, drain_prev=True, prefetch=True)
        do_block(2 * i + 2, 0, drain_prev=True, prefetch=True)
        return carry

      lax.fori_loop(0, (nblocks - 3) // 2, body, 0)
      do_block(nblocks - 2, 1, drain_prev=True, prefetch=True)
      do_block(nblocks - 1, 0, drain_prev=True, prefetch=False)
      for b in range(K):
        wait_scatter(0, b)
    else:
      def issue_block(i):
        for b in range(K):
          issue_scatter(i * K + b, 0, b)

      def drain_block():
        for b in range(K):
          wait_scatter(0, b)

      issue_block(0)

      def body(i, carry):
        issue_block(i + 1)
        drain_block()
        return carry

      lax.fori_loop(0, nblocks - 1, body, 0)
      drain_block()

    plsc.subcore_barrier()
    pltpu.sync_copy(acc.at[pl.ds(s * rpt8, rpt8)],
                    out.at[c, pl.ds(s * rpt8, rpt8)])

    @pl.when(s == _NS - 1)
    def _():
      pltpu.sync_copy(acc.at[pl.ds(_NS * rpt8, tail)],
                      out.at[c, pl.ds(_NS * rpt8, tail)])

  return prop


def _dinv_of(dp):
  deg = dp[0][:, 0:1] + dp[1][:, 0:1]
  return jnp.where(deg > 0, lax.rsqrt(jnp.maximum(deg, 1e-12)), 0.0)


def _tc_input(x, w1, v1):
  n, f = x.shape
  kd = w1.shape[1]

  def body(x_ref, w_ref, v_ref, z_ref, r_ref):
    xb = x_ref[...]
    rn = xb / jnp.maximum(jnp.sum(xb, axis=1, keepdims=True), 1e-8)
    z_ref[...] = jnp.dot(rn, w_ref[...], preferred_element_type=jnp.float32)
    r_ref[...] = jnp.dot(rn, v_ref[...], preferred_element_type=jnp.float32)

  return pl.pallas_call(
      body,
      grid=(n // _RT,),
      in_specs=[pl.BlockSpec((_RT, f), lambda i: (i, 0)),
                pl.BlockSpec((f, kd), lambda i: (0, 0)),
                pl.BlockSpec((f, kd), lambda i: (0, 0))],
      out_specs=[pl.BlockSpec((_RT, kd), lambda i: (i, 0)),
                 pl.BlockSpec((_RT, kd), lambda i: (i, 0))],
      out_shape=[jax.ShapeDtypeStruct((n, kd), jnp.float32),
                 jax.ShapeDtypeStruct((n, kd), jnp.float32)],
  )(x, w1, v1)


def _tc_scale(degp, z):
  n, kd = z.shape

  def body(dp_ref, z_ref, o_ref):
    o_ref[...] = z_ref[...] * _dinv_of(dp_ref[...])

  return pl.pallas_call(
      body,
      grid=(n // _RT,),
      in_specs=[pl.BlockSpec((_NC, _RT, 16), lambda i: (0, i, 0)),
                pl.BlockSpec((_RT, kd), lambda i: (i, 0))],
      out_specs=pl.BlockSpec((_RT, kd), lambda i: (i, 0)),
      out_shape=jax.ShapeDtypeStruct((n, kd), jnp.float32),
  )(degp, z)


def _tc_combine1(degp, pp, r1, b1, v2):
  n = r1.shape[0]
  kd = r1.shape[1]
  hid = kd // 2
  dout = v2.shape[1]

  def body(dp_ref, pp_ref, r1_ref, b1_ref, v2_ref, hs_ref, hv2_ref):
    dinv = _dinv_of(dp_ref[...])
    agg = (pp_ref[0] + pp_ref[1]) * dinv
    u = agg + r1_ref[...] + b1_ref[...]
    h = 0.5 * (jax.nn.relu(u[:, :hid]) + jax.nn.relu(u[:, hid:]))
    hs_ref[...] = h * dinv
    hv2_ref[...] = jnp.dot(h, v2_ref[...], preferred_element_type=jnp.float32)

  return pl.pallas_call(
      body,
      grid=(n // _RT,),
      in_specs=[pl.BlockSpec((_NC, _RT, 16), lambda i: (0, i, 0)),
                pl.BlockSpec((_NC, _RT, kd), lambda i: (0, i, 0)),
                pl.BlockSpec((_RT, kd), lambda i: (i, 0)),
                pl.BlockSpec((1, kd), lambda i: (0, 0)),
                pl.BlockSpec((hid, dout), lambda i: (0, 0))],
      out_specs=[pl.BlockSpec((_RT, hid), lambda i: (i, 0)),
                 pl.BlockSpec((_RT, dout), lambda i: (i, 0))],
      out_shape=[jax.ShapeDtypeStruct((n, hid), jnp.float32),
                 jax.ShapeDtypeStruct((n, dout), jnp.float32)],
  )(degp, pp, r1, b1, v2)


def _tc_combine2(degp, qp, hv2, w2, b2):
  n, dout = hv2.shape
  hid = w2.shape[0]

  def body(dp_ref, qp_ref, hv2_ref, w2_ref, b2_ref, o_ref):
    dinv = _dinv_of(dp_ref[...])
    aggh = (qp_ref[0] + qp_ref[1]) * dinv
    o_ref[...] = jax.nn.relu(
        jnp.dot(aggh, w2_ref[...], preferred_element_type=jnp.float32)
        + hv2_ref[...] + b2_ref[...])

  return pl.pallas_call(
      body,
      grid=(n // _RT,),
      in_specs=[pl.BlockSpec((_NC, _RT, 16), lambda i: (0, i, 0)),
                pl.BlockSpec((_NC, _RT, hid), lambda i: (0, i, 0)),
                pl.BlockSpec((_RT, dout), lambda i: (i, 0)),
                pl.BlockSpec((hid, dout), lambda i: (0, 0)),
                pl.BlockSpec((1, dout), lambda i: (0, 0))],
      out_specs=pl.BlockSpec((_RT, dout), lambda i: (i, 0)),
      out_shape=jax.ShapeDtypeStruct((n, dout), jnp.float32),
  )(degp, qp, hv2, w2, b2)


def kernel(x, edge_index, c1_init, c1_root, c1_bias, c2_init, c2_root, c2_bias):
  n, f = x.shape
  e = edge_index.shape[1]
  k1, hid = c1_init.shape[0], c1_init.shape[2]
  dout = c2_init.shape[2]
  kd = k1 * hid

  row2d = edge_index[0].reshape(e // _CH, _CH)
  col2d = edge_index[1].reshape(e // _CH, _CH)
  w1 = jnp.transpose(c1_init, (1, 0, 2)).reshape(f, kd)
  v1 = jnp.transpose(c1_root[0], (1, 0, 2)).reshape(f, kd)
  b1 = jnp.transpose(c1_bias[0], (1, 0, 2)).reshape(1, kd)
  w2 = c2_init[0]
  v2 = c2_root[0, 0]
  b2 = c2_bias[0, 0]
  zeros16 = jnp.zeros((n, 16), jnp.float32)
  zeros32 = jnp.zeros((n, kd), jnp.float32)
  ones16 = jnp.ones((_CH, 16), jnp.float32)

  z, r1 = _tc_input(x, w1, v1)
  degp = _propagate(n, e, 16, False)(row2d, col2d, zeros16, zeros16, ones16)
  zs = _tc_scale(degp, z)
  pp = _propagate(n, e, kd, True)(row2d, col2d, zs, zeros32, ones16)
  hs, hv2 = _tc_combine1(degp, pp, r1, b1, v2)
  qp = _propagate(n, e, hid, True)(row2d, col2d, hs, zeros16, ones16)
  return _tc_combine2(degp, qp, hv2, w2, b2)


# speedup vs baseline: 181.5960x; 1.0397x over previous
import functools

import jax
import jax.numpy as jnp
from jax import lax
from jax.experimental import pallas as pl
from jax.experimental.pallas import tpu as pltpu
from jax.experimental.pallas import tpu_sc as plsc

_NC = 2
_NS = 16
_NW = _NC * _NS
_CH = 80
_RT = 400


def _propagate(n, e, d, gather):
  epw = e // _NW
  nchunks = epw // _CH
  K = 5
  nblocks = nchunks // K
  rpt8 = (n // _NS) // 8 * 8
  tail = n - _NS * rpt8
  mesh = plsc.VectorSubcoreMesh(core_axis_name="c", subcore_axis_name="s",
                                num_cores=_NC, num_subcores=_NS)

  @functools.partial(
      pl.kernel,
      out_type=jax.ShapeDtypeStruct((_NC, n, d), jnp.float32),
      mesh=mesh,
      compiler_params=pltpu.CompilerParams(use_tc_tiling_on_sc=False),
      scratch_types=[
          pltpu.VMEM((nchunks, _CH), jnp.int32),
          pltpu.VMEM((nchunks, _CH), jnp.int32),
          pltpu.VMEM((2, K, _CH, d), jnp.float32),
          pltpu.VMEM_SHARED((n, d), jnp.float32),
          pltpu.VMEM_SHARED((n, d), jnp.float32),
          pltpu.SemaphoreType.DMA,
          pltpu.SemaphoreType.DMA,
      ],
  )
  def prop(row2d, col2d, table, zeros, ones, out,
           ridx, cidx, bufs, acc, tbl, gsem, ssem):
    c = lax.axis_index("c")
    s = lax.axis_index("s")
    wid = s * _NC + c
    pltpu.sync_copy(col2d.at[pl.ds(wid * nchunks, nchunks)], cidx)
    if gather:
      pltpu.sync_copy(row2d.at[pl.ds(wid * nchunks, nchunks)], ridx)
    pltpu.sync_copy(zeros.at[pl.ds(s * rpt8, rpt8)],
                    acc.at[pl.ds(s * rpt8, rpt8)])

    @pl.when(s == _NS - 1)
    def _():
      pltpu.sync_copy(zeros.at[pl.ds(_NS * rpt8, tail)],
                      acc.at[pl.ds(_NS * rpt8, tail)])

    if gather:
      pltpu.sync_copy(table.at[pl.ds(s * rpt8, rpt8)],
                      tbl.at[pl.ds(s * rpt8, rpt8)])

      @pl.when(s == _NS - 1)
      def _():
        pltpu.sync_copy(table.at[pl.ds(_NS * rpt8, tail)],
                        tbl.at[pl.ds(_NS * rpt8, tail)])

    if not gather:
      for b in range(K):
        pltpu.sync_copy(ones, bufs.at[0, b])
    plsc.subcore_barrier()

    def issue_gather(j, g, b):
      pltpu.async_copy(tbl.at[ridx.at[j]], bufs.at[g, b], gsem)

    def wait_gather(g, b):
      pltpu.make_async_copy(table.at[pl.ds(0, _CH)], bufs.at[g, b], gsem).wait()

    def issue_scatter(j, g, b):
      pltpu.async_copy(bufs.at[g, b], acc.at[cidx.at[j]], ssem, add=True)

    def wait_scatter(g, b):
      pltpu.make_async_copy(table.at[pl.ds(0, _CH)], bufs.at[g, b], ssem).wait()

    if gather:
      def do_block(bi, g, drain_prev, prefetch):
        if drain_prev:
          for b in range(K):
            wait_scatter(1 - g, b)
        for b in range(K):
          wait_gather(g, b)
        if prefetch:
          for b in range(K):
            issue_gather((bi + 1) * K + b, 1 - g, b)
        for b in range(K):
          issue_scatter(bi * K + b, g, b)

      for b in range(K):
        issue_gather(b, 0, b)
      do_block(0, 0, drain_prev=False, prefetch=True)

      def body(i, carry):
        do_block(2 * i + 1, 1, drain_prev=True, prefetch=True)
        do_block(2 * i + 2, 0, drain_prev=True, prefetch=True)
        return carry

      lax.fori_loop(0, (nblocks - 3) // 2, body, 0)
      do_block(nblocks - 2, 1, drain_prev=True, prefetch=True)
      do_block(nblocks - 1, 0, drain_prev=True, prefetch=False)
      for b in range(K):
        wait_scatter(0, b)
    else:
      def issue_block(i):
        for b in range(K):
          issue_scatter(i * K + b, 0, b)

      def drain_block():
        for b in range(K):
          wait_scatter(0, b)

      issue_block(0)

      def body(i, carry):
        issue_block(i + 1)
        drain_block()
        return carry

      lax.fori_loop(0, nblocks - 1, body, 0)
      drain_block()

    plsc.subcore_barrier()
    pltpu.sync_copy(acc.at[pl.ds(s * rpt8, rpt8)],
                    out.at[c, pl.ds(s * rpt8, rpt8)])

    @pl.when(s == _NS - 1)
    def _():
      pltpu.sync_copy(acc.at[pl.ds(_NS * rpt8, tail)],
                      out.at[c, pl.ds(_NS * rpt8, tail)])

  return prop


def _dinv_of(dp):
  deg = dp[0][:, 0:1] + dp[1][:, 0:1]
  return jnp.where(deg > 0, lax.rsqrt(jnp.maximum(deg, 1e-12)), 0.0)


def _tc_input(x, w1, v1):
  n, f = x.shape
  kd = w1.shape[1]

  def body(x_ref, w_ref, v_ref, z_ref, r_ref):
    xb = x_ref[...]
    rn = xb / jnp.maximum(jnp.sum(xb, axis=1, keepdims=True), 1e-8)
    z_ref[...] = jnp.dot(rn, w_ref[...], preferred_element_type=jnp.float32)
    r_ref[...] = jnp.dot(rn, v_ref[...], preferred_element_type=jnp.float32)

  return pl.pallas_call(
      body,
      grid=(n // _RT,),
      in_specs=[pl.BlockSpec((_RT, f), lambda i: (i, 0)),
                pl.BlockSpec((f, kd), lambda i: (0, 0)),
                pl.BlockSpec((f, kd), lambda i: (0, 0))],
      out_specs=[pl.BlockSpec((_RT, kd), lambda i: (i, 0)),
                 pl.BlockSpec((_RT, kd), lambda i: (i, 0))],
      out_shape=[jax.ShapeDtypeStruct((n, kd), jnp.float32),
                 jax.ShapeDtypeStruct((n, kd), jnp.float32)],
  )(x, w1, v1)


def _tc_scale(degp, z):
  n, kd = z.shape

  def body(dp_ref, z_ref, o_ref):
    o_ref[...] = z_ref[...] * _dinv_of(dp_ref[...])

  return pl.pallas_call(
      body,
      grid=(n // _RT,),
      in_specs=[pl.BlockSpec((_NC, _RT, 16), lambda i: (0, i, 0)),
                pl.BlockSpec((_RT, kd), lambda i: (i, 0))],
      out_specs=pl.BlockSpec((_RT, kd), lambda i: (i, 0)),
      out_shape=jax.ShapeDtypeStruct((n, kd), jnp.float32),
  )(degp, z)


def _tc_combine1(degp, pp, r1, b1, v2):
  n = r1.shape[0]
  kd = r1.shape[1]
  hid = kd // 2
  dout = v2.shape[1]

  def body(dp_ref, pp_ref, r1_ref, b1_ref, v2_ref, hs_ref, hv2_ref):
    dinv = _dinv_of(dp_ref[...])
    agg = (pp_ref[0] + pp_ref[1]) * dinv
    u = agg + r1_ref[...] + b1_ref[...]
    h = 0.5 * (jax.nn.relu(u[:, :hid]) + jax.nn.relu(u[:, hid:]))
    hs_ref[...] = h * dinv
    hv2_ref[...] = jnp.dot(h, v2_ref[...], preferred_element_type=jnp.float32)

  return pl.pallas_call(
      body,
      grid=(n // _RT,),
      in_specs=[pl.BlockSpec((_NC, _RT, 16), lambda i: (0, i, 0)),
                pl.BlockSpec((_NC, _RT, kd), lambda i: (0, i, 0)),
                pl.BlockSpec((_RT, kd), lambda i: (i, 0)),
                pl.BlockSpec((1, kd), lambda i: (0, 0)),
                pl.BlockSpec((hid, dout), lambda i: (0, 0))],
      out_specs=[pl.BlockSpec((_RT, hid), lambda i: (i, 0)),
                 pl.BlockSpec((_RT, dout), lambda i: (i, 0))],
      out_shape=[jax.ShapeDtypeStruct((n, hid), jnp.float32),
                 jax.ShapeDtypeStruct((n, dout), jnp.float32)],
  )(degp, pp, r1, b1, v2)


def _tc_combine2(degp, qp, hv2, w2, b2):
  n, dout = hv2.shape
  hid = w2.shape[0]

  def body(dp_ref, qp_ref, hv2_ref, w2_ref, b2_ref, o_ref):
    dinv = _dinv_of(dp_ref[...])
    aggh = (qp_ref[0] + qp_ref[1]) * dinv
    o_ref[...] = jax.nn.relu(
        jnp.dot(aggh, w2_ref[...], preferred_element_type=jnp.float32)
        + hv2_ref[...] + b2_ref[...])

  return pl.pallas_call(
      body,
      grid=(n // _RT,),
      in_specs=[pl.BlockSpec((_NC, _RT, 16), lambda i: (0, i, 0)),
                pl.BlockSpec((_NC, _RT, hid), lambda i: (0, i, 0)),
                pl.BlockSpec((_RT, dout), lambda i: (i, 0)),
                pl.BlockSpec((hid, dout), lambda i: (0, 0)),
                pl.BlockSpec((1, dout), lambda i: (0, 0))],
      out_specs=pl.BlockSpec((_RT, dout), lambda i: (i, 0)),
      out_shape=jax.ShapeDtypeStruct((n, dout), jnp.float32),
  )(degp, qp, hv2, w2, b2)


def kernel(x, edge_index, c1_init, c1_root, c1_bias, c2_init, c2_root, c2_bias):
  n, f = x.shape
  e = edge_index.shape[1]
  k1, hid = c1_init.shape[0], c1_init.shape[2]
  dout = c2_init.shape[2]
  kd = k1 * hid

  row2d = edge_index[0].reshape(e // _CH, _CH)
  col2d = edge_index[1].reshape(e // _CH, _CH)
  w1 = jnp.transpose(c1_init, (1, 0, 2)).reshape(f, kd)
  v1 = jnp.transpose(c1_root[0], (1, 0, 2)).reshape(f, kd)
  b1 = jnp.transpose(c1_bias[0], (1, 0, 2)).reshape(1, kd)
  w2 = c2_init[0]
  v2 = c2_root[0, 0]
  b2 = c2_bias[0, 0]
  zeros16 = jnp.zeros((n, 16), jnp.float32)
  zeros32 = jnp.zeros((n, kd), jnp.float32)
  ones16 = jnp.ones((_CH, 16), jnp.float32)

  z, r1 = _tc_input(x, w1, v1)
  degp = _propagate(n, e, 16, False)(row2d, col2d, zeros16, zeros16, ones16)
  zs = _tc_scale(degp, z)
  pp = _propagate(n, e, kd, True)(row2d, col2d, zs, zeros32, ones16)
  hs, hv2 = _tc_combine1(degp, pp, r1, b1, v2)
  qp = _propagate(n, e, hid, True)(row2d, col2d, hs, zeros16, ones16)
  return _tc_combine2(degp, qp, hv2, w2, b2)
